# baseline (speedup 1.0000x reference)
import functools

import jax
import jax.numpy as jnp
from jax import lax
from jax.experimental import pallas as pl
from jax.experimental.pallas import tpu as pltpu
from jax.experimental.pallas import tpu_sc as plsc

VOCAB_SIZE = 1000000
EMBED_DIM = 64
NUM_SAMPLED = 64

CHUNK = 128
LANES = 16


@functools.lru_cache(maxsize=None)
def _make_gather(batch):
    info = plsc.get_sparse_core_info()
    nc, ns = info.num_cores, info.num_subcores
    nw = nc * ns
    b_per_w = batch // nw
    assert b_per_w % CHUNK == 0
    n_chunks = b_per_w // CHUNK
    n_steps = NUM_SAMPLED * n_chunks
    n_bj = batch // CHUNK

    mesh = plsc.VectorSubcoreMesh(core_axis_name="c", subcore_axis_name="s")

    @functools.partial(
        pl.kernel,
        mesh=mesh,
        out_type=jax.ShapeDtypeStruct(
            (NUM_SAMPLED, EMBED_DIM // 8, n_bj, 8, CHUNK), jnp.float32
        ),
        compiler_params=pltpu.CompilerParams(
            use_tc_tiling_on_sc=False, needs_layout_passes=False
        ),
        scratch_types=[
            pltpu.VMEM((CHUNK,), jnp.int32),
            pltpu.VMEM((CHUNK,), jnp.int32),
            pltpu.VMEM((CHUNK,), jnp.int32),
            pltpu.VMEM((CHUNK,), jnp.int32),
            pltpu.VMEM((CHUNK, EMBED_DIM), jnp.float32),
            pltpu.VMEM((CHUNK, EMBED_DIM), jnp.float32),
            pltpu.VMEM((EMBED_DIM // 8, 8, CHUNK), jnp.float32),
            pltpu.VMEM((EMBED_DIM // 8, 8, CHUNK), jnp.float32),
            pltpu.SemaphoreType.DMA,
            pltpu.SemaphoreType.DMA,
            pltpu.SemaphoreType.DMA,
            pltpu.SemaphoreType.DMA,
        ],
    )
    def gather_kernel(
        tablep_hbm, idxt_hbm, out_hbm,
        idx0, idx1, pr0, pr1, rows0, rows1, trans0, trans1,
        gs0, gs1, os0, os1,
    ):
        wid = lax.axis_index("s") * nc + lax.axis_index("c")
        b0 = wid * b_per_w
        j0 = wid * n_chunks
        idxs = (idx0, idx1)
        prs = (pr0, pr1)
        rows = (rows0, rows1)
        transs = (trans0, trans1)
        gsems = (gs0, gs1)
        osems = (os0, os1)
        iota = lax.iota(jnp.int32, LANES)

        def build_and_fire(buf, s, c):
            pltpu.sync_copy(
                idxt_hbm.at[s, pl.ds(b0 + c * CHUNK, CHUNK)], idxs[buf]
            )
            for g in range(CHUNK // LANES):
                iv = idxs[buf][pl.ds(g * LANES, LANES)]
                prs[buf][pl.ds(g * LANES, LANES)] = lax.shift_left(iv, 1)
            pltpu.async_copy(tablep_hbm.at[prs[buf]], rows[buf], gsems[buf])

        def drain_gather(buf):
            pltpu.make_async_copy(
                tablep_hbm.at[pl.ds(0, CHUNK)], rows[buf], gsems[buf]
            ).wait()

        def wait_out(buf):
            pltpu.make_async_copy(
                out_hbm.at[0, :, 0], transs[buf], osems[buf]
            ).wait()

        def extract(buf):
            def kg_body(kg, carry):
                k0 = kg * LANES
                rowv = k0 + iota
                for e in range(EMBED_DIM):
                    ev = jnp.full((LANES,), e, jnp.int32)
                    vals = plsc.load_gather(rows[buf], [rowv, ev])
                    transs[buf][e // 8, e % 8, pl.ds(k0, LANES)] = vals
                return carry

            lax.fori_loop(0, CHUNK // LANES, kg_body, 0, unroll=False)

        def write_out(buf, s, c):
            pltpu.async_copy(
                transs[buf], out_hbm.at[s, :, j0 + c], osems[buf]
            )

        build_and_fire(0, 0, 0)

        def pair_body(t2, carry):
            for buf in range(2):
                t = 2 * t2 + buf
                s = t // n_chunks
                c = lax.rem(t, n_chunks)
                tn = t + 1
                sn = tn // n_chunks
                cn = lax.rem(tn, n_chunks)

                @pl.when(t < n_steps - 1)
                def _():
                    build_and_fire(1 - buf, sn, cn)

                drain_gather(buf)

                @pl.when(t2 >= 1)
                def _():
                    wait_out(buf)

                extract(buf)
                write_out(buf, s, c)
            return carry

        lax.fori_loop(0, n_steps // 2, pair_body, 0, unroll=False)

        wait_out(0)
        wait_out(1)

    return gather_kernel


def kernel(target_index, embedding_table):
    batch = target_index.shape[0]
    skey = jax.random.key(42)
    sampled_idx = jax.random.randint(
        skey, (batch, NUM_SAMPLED), 1, VOCAB_SIZE, dtype=jnp.int32
    )
    idxt = jnp.transpose(sampled_idx)
    tablep = jnp.pad(embedding_table, ((0, 0), (0, EMBED_DIM))).reshape(
        2 * VOCAB_SIZE, EMBED_DIM
    )
    gather_kernel = _make_gather(batch)
    out5 = gather_kernel(tablep, idxt)
    return jnp.transpose(out5, (2, 4, 0, 1, 3)).reshape(
        batch, NUM_SAMPLED, EMBED_DIM
    )


# speedup vs baseline: 1.5432x; 1.5432x over previous
import functools

import jax
import jax.numpy as jnp
from jax import lax
from jax.experimental import pallas as pl
from jax.experimental.pallas import tpu as pltpu
from jax.experimental.pallas import tpu_sc as plsc

VOCAB_SIZE = 1000000
EMBED_DIM = 64
NUM_SAMPLED = 64

CHUNK = 128
LANES = 16


@functools.lru_cache(maxsize=None)
def _make_gather(batch):
    info = plsc.get_sparse_core_info()
    nc, ns = info.num_cores, info.num_subcores
    nw = nc * ns
    b_per_w = batch // nw
    assert b_per_w % CHUNK == 0
    n_chunks = b_per_w // CHUNK
    n_steps = NUM_SAMPLED * n_chunks
    n_bj = batch // CHUNK

    mesh = plsc.VectorSubcoreMesh(core_axis_name="c", subcore_axis_name="s")

    @functools.partial(
        pl.kernel,
        mesh=mesh,
        out_type=jax.ShapeDtypeStruct(
            (NUM_SAMPLED, EMBED_DIM // 8, n_bj, 8, CHUNK), jnp.float32
        ),
        compiler_params=pltpu.CompilerParams(
            use_tc_tiling_on_sc=False, needs_layout_passes=False
        ),
        scratch_types=[
            pltpu.VMEM((CHUNK,), jnp.int32),
            pltpu.VMEM((CHUNK,), jnp.int32),
            pltpu.VMEM((CHUNK,), jnp.int32),
            pltpu.VMEM((CHUNK,), jnp.int32),
            pltpu.VMEM((CHUNK, EMBED_DIM), jnp.float32),
            pltpu.VMEM((CHUNK, EMBED_DIM), jnp.float32),
            pltpu.VMEM((EMBED_DIM, CHUNK), jnp.float32),
            pltpu.VMEM((EMBED_DIM, CHUNK), jnp.float32),
            pltpu.SemaphoreType.DMA,
            pltpu.SemaphoreType.DMA,
            pltpu.SemaphoreType.DMA,
            pltpu.SemaphoreType.DMA,
        ],
    )
    def gather_kernel(
        tablep_hbm, idxt_hbm, out_hbm,
        idx0, idx1, pr0, pr1, rows0, rows1, trans0, trans1,
        gs0, gs1, os0, os1,
    ):
        wid = lax.axis_index("s") * nc + lax.axis_index("c")
        b0 = wid * b_per_w
        j0 = wid * n_chunks
        idxs = (idx0, idx1)
        prs = (pr0, pr1)
        rows = (rows0, rows1)
        transs = (trans0, trans1)
        gsems = (gs0, gs1)
        osems = (os0, os1)
        iota = lax.iota(jnp.int32, LANES)

        def build_and_fire(buf, s, c):
            pltpu.sync_copy(
                idxt_hbm.at[s, pl.ds(b0 + c * CHUNK, CHUNK)], idxs[buf]
            )
            for g in range(CHUNK // LANES):
                iv = idxs[buf][pl.ds(g * LANES, LANES)]
                prs[buf][pl.ds(g * LANES, LANES)] = lax.shift_left(iv, 1)
            pltpu.async_copy(tablep_hbm.at[prs[buf]], rows[buf], gsems[buf])

        def drain_gather(buf):
            pltpu.make_async_copy(
                tablep_hbm.at[pl.ds(0, CHUNK)], rows[buf], gsems[buf]
            ).wait()

        def wait_out(buf):
            for i in range(EMBED_DIM // 8):
                pltpu.make_async_copy(
                    out_hbm.at[0, i, 0],
                    transs[buf].at[pl.ds(8 * i, 8), :],
                    osems[buf],
                ).wait()

        rots = [
            jnp.bitwise_and(iota + j, LANES - 1) for j in range(LANES)
        ]

        def extract(buf):
            def kg_body(kg, carry):
                k0 = kg * LANES
                rowv = k0 + iota
                for e0 in range(0, EMBED_DIM, LANES):
                    for j in range(LANES):
                        erv = e0 + rots[j]
                        vals = plsc.load_gather(rows[buf], [rowv, erv])
                        plsc.store_scatter(transs[buf], [erv, rowv], vals)
                return carry

            lax.fori_loop(0, CHUNK // LANES, kg_body, 0, unroll=False)

        def write_out(buf, s, c):
            for i in range(EMBED_DIM // 8):
                pltpu.async_copy(
                    transs[buf].at[pl.ds(8 * i, 8), :],
                    out_hbm.at[s, i, j0 + c],
                    osems[buf],
                )

        build_and_fire(0, 0, 0)

        def pair_body(t2, carry):
            for buf in range(2):
                t = 2 * t2 + buf
                s = t // n_chunks
                c = lax.rem(t, n_chunks)
                tn = t + 1
                sn = tn // n_chunks
                cn = lax.rem(tn, n_chunks)

                @pl.when(t < n_steps - 1)
                def _():
                    build_and_fire(1 - buf, sn, cn)

                drain_gather(buf)

                @pl.when(t2 >= 1)
                def _():
                    wait_out(buf)

                extract(buf)
                write_out(buf, s, c)
            return carry

        lax.fori_loop(0, n_steps // 2, pair_body, 0, unroll=False)

        wait_out(0)
        wait_out(1)

    return gather_kernel


def kernel(target_index, embedding_table):
    batch = target_index.shape[0]
    skey = jax.random.key(42)
    sampled_idx = jax.random.randint(
        skey, (batch, NUM_SAMPLED), 1, VOCAB_SIZE, dtype=jnp.int32
    )
    idxt = jnp.transpose(sampled_idx)
    tablep = jnp.pad(embedding_table, ((0, 0), (0, EMBED_DIM))).reshape(
        2 * VOCAB_SIZE, EMBED_DIM
    )
    gather_kernel = _make_gather(batch)
    out5 = gather_kernel(tablep, idxt)
    return jnp.transpose(out5, (2, 4, 0, 1, 3)).reshape(
        batch, NUM_SAMPLED, EMBED_DIM
    )


# speedup vs baseline: 1.6809x; 1.0893x over previous
import functools

import jax
import jax.numpy as jnp
from jax import lax
from jax.experimental import pallas as pl
from jax.experimental.pallas import tpu as pltpu
from jax.experimental.pallas import tpu_sc as plsc

VOCAB_SIZE = 1000000
EMBED_DIM = 64
NUM_SAMPLED = 64

CHUNK = 128
LANES = 16


@functools.lru_cache(maxsize=None)
def _make_gather(batch):
    info = plsc.get_sparse_core_info()
    nc, ns = info.num_cores, info.num_subcores
    nw = nc * ns
    b_per_w = batch // nw
    assert b_per_w % CHUNK == 0
    n_chunks = b_per_w // CHUNK
    n_steps = NUM_SAMPLED * n_chunks
    n_bj = batch // CHUNK

    mesh = plsc.VectorSubcoreMesh(core_axis_name="c", subcore_axis_name="s")

    @functools.partial(
        pl.kernel,
        mesh=mesh,
        out_type=jax.ShapeDtypeStruct(
            (NUM_SAMPLED, EMBED_DIM // 8, n_bj, 8, CHUNK), jnp.float32
        ),
        compiler_params=pltpu.CompilerParams(
            use_tc_tiling_on_sc=False, needs_layout_passes=False
        ),
        scratch_types=[
            pltpu.VMEM((NUM_SAMPLED, b_per_w), jnp.int32),
            pltpu.VMEM((CHUNK,), jnp.int32),
            pltpu.VMEM((CHUNK,), jnp.int32),
            pltpu.VMEM((CHUNK, EMBED_DIM), jnp.float32),
            pltpu.VMEM((CHUNK, EMBED_DIM), jnp.float32),
            pltpu.VMEM((EMBED_DIM, CHUNK), jnp.float32),
            pltpu.VMEM((EMBED_DIM, CHUNK), jnp.float32),
            pltpu.SemaphoreType.DMA,
            pltpu.SemaphoreType.DMA,
            pltpu.SemaphoreType.DMA,
            pltpu.SemaphoreType.DMA,
        ],
    )
    def gather_kernel(
        tablep_hbm, idxt_hbm, out_hbm,
        idxall, pr0, pr1, rows0, rows1, trans0, trans1,
        gs0, gs1, os0, os1,
    ):
        wid = lax.axis_index("s") * nc + lax.axis_index("c")
        b0 = wid * b_per_w
        j0 = wid * n_chunks
        prs = (pr0, pr1)
        rows = (rows0, rows1)
        transs = (trans0, trans1)
        gsems = (gs0, gs1)
        osems = (os0, os1)
        iota = lax.iota(jnp.int32, LANES)

        def build_and_fire(buf, s, c):
            for g in range(CHUNK // LANES):
                iv = idxall[s, pl.ds(c * CHUNK + g * LANES, LANES)]
                prs[buf][pl.ds(g * LANES, LANES)] = lax.shift_left(iv, 1)
            pltpu.async_copy(tablep_hbm.at[prs[buf]], rows[buf], gsems[buf])

        def drain_gather(buf):
            pltpu.make_async_copy(
                tablep_hbm.at[pl.ds(0, CHUNK)], rows[buf], gsems[buf]
            ).wait()

        def wait_out(buf):
            for i in range(EMBED_DIM // 8):
                pltpu.make_async_copy(
                    out_hbm.at[0, i, 0],
                    transs[buf].at[pl.ds(8 * i, 8), :],
                    osems[buf],
                ).wait()

        rots = [
            jnp.bitwise_and(iota + j, LANES - 1) for j in range(LANES)
        ]

        def extract(buf):
            def kg_body(kg, carry):
                k0 = kg * LANES
                rowv = k0 + iota
                for e0 in range(0, EMBED_DIM, LANES):
                    for j in range(LANES):
                        erv = e0 + rots[j]
                        vals = plsc.load_gather(rows[buf], [rowv, erv])
                        plsc.store_scatter(transs[buf], [erv, rowv], vals)
                return carry

            lax.fori_loop(0, CHUNK // LANES, kg_body, 0, unroll=False)

        def write_out(buf, s, c):
            for i in range(EMBED_DIM // 8):
                pltpu.async_copy(
                    transs[buf].at[pl.ds(8 * i, 8), :],
                    out_hbm.at[s, i, j0 + c],
                    osems[buf],
                )

        pltpu.sync_copy(idxt_hbm.at[:, pl.ds(b0, b_per_w)], idxall)

        build_and_fire(0, 0, 0)

        def pair_body(t2, carry):
            for buf in range(2):
                t = 2 * t2 + buf
                s = t // n_chunks
                c = lax.rem(t, n_chunks)
                tn = t + 1
                sn = tn // n_chunks
                cn = lax.rem(tn, n_chunks)

                @pl.when(t < n_steps - 1)
                def _():
                    build_and_fire(1 - buf, sn, cn)

                drain_gather(buf)

                @pl.when(t2 >= 1)
                def _():
                    wait_out(buf)

                extract(buf)
                write_out(buf, s, c)
            return carry

        lax.fori_loop(0, n_steps // 2, pair_body, 0, unroll=False)

        wait_out(0)
        wait_out(1)

    return gather_kernel


def kernel(target_index, embedding_table):
    batch = target_index.shape[0]
    skey = jax.random.key(42)
    sampled_idx = jax.random.randint(
        skey, (batch, NUM_SAMPLED), 1, VOCAB_SIZE, dtype=jnp.int32
    )
    idxt = jnp.transpose(sampled_idx)
    tablep = jnp.pad(embedding_table, ((0, 0), (0, EMBED_DIM))).reshape(
        2 * VOCAB_SIZE, EMBED_DIM
    )
    gather_kernel = _make_gather(batch)
    out5 = gather_kernel(tablep, idxt)
    return jnp.transpose(out5, (2, 4, 0, 1, 3)).reshape(
        batch, NUM_SAMPLED, EMBED_DIM
    )


# speedup vs baseline: 2.6644x; 1.5851x over previous
import functools

import jax
import jax.numpy as jnp
from jax import lax
from jax.experimental import pallas as pl
from jax.experimental.pallas import tpu as pltpu
from jax.experimental.pallas import tpu_sc as plsc

VOCAB_SIZE = 1000000
EMBED_DIM = 64
NUM_SAMPLED = 64

CHUNK = 128
LANES = 16


@functools.lru_cache(maxsize=None)
def _make_gather(batch):
    info = plsc.get_sparse_core_info()
    nc, ns = info.num_cores, info.num_subcores
    nw = nc * ns
    b_per_w = batch // nw
    assert b_per_w % CHUNK == 0
    n_chunks = b_per_w // CHUNK
    n_steps = NUM_SAMPLED * n_chunks
    n_bj = batch // CHUNK

    mesh = plsc.VectorSubcoreMesh(core_axis_name="c", subcore_axis_name="s")

    @functools.partial(
        pl.kernel,
        mesh=mesh,
        out_type=jax.ShapeDtypeStruct(
            (NUM_SAMPLED, EMBED_DIM // 8, n_bj, 8, CHUNK), jnp.float32
        ),
        compiler_params=pltpu.CompilerParams(
            use_tc_tiling_on_sc=False, needs_layout_passes=False
        ),
        scratch_types=[
            pltpu.VMEM((NUM_SAMPLED, b_per_w), jnp.int32),
            pltpu.VMEM((CHUNK,), jnp.int32),
            pltpu.VMEM((CHUNK,), jnp.int32),
            pltpu.VMEM((CHUNK, EMBED_DIM), jnp.float32),
            pltpu.VMEM((CHUNK, EMBED_DIM), jnp.float32),
            pltpu.VMEM((EMBED_DIM, CHUNK), jnp.float32),
            pltpu.VMEM((EMBED_DIM, CHUNK), jnp.float32),
            pltpu.SemaphoreType.DMA,
            pltpu.SemaphoreType.DMA,
            pltpu.SemaphoreType.DMA,
            pltpu.SemaphoreType.DMA,
        ],
    )
    def gather_kernel(
        tablep_hbm, idxt_hbm, out_hbm,
        idxall, pr0, pr1, rows0, rows1, trans0, trans1,
        gs0, gs1, os0, os1,
    ):
        wid = lax.axis_index("s") * nc + lax.axis_index("c")
        b0 = wid * b_per_w
        j0 = wid * n_chunks
        prs = (pr0, pr1)
        rows = (rows0, rows1)
        transs = (trans0, trans1)
        gsems = (gs0, gs1)
        osems = (os0, os1)
        iota = lax.iota(jnp.int32, LANES)

        def build_and_fire(buf, s, c):
            for g in range(CHUNK // LANES):
                iv = idxall[s, pl.ds(c * CHUNK + g * LANES, LANES)]
                prs[buf][pl.ds(g * LANES, LANES)] = lax.shift_left(iv, 1)
            pltpu.async_copy(tablep_hbm.at[prs[buf]], rows[buf], gsems[buf])

        def drain_gather(buf):
            pltpu.make_async_copy(
                tablep_hbm.at[pl.ds(0, CHUNK)], rows[buf], gsems[buf]
            ).wait()

        def wait_out(buf):
            for i in range(EMBED_DIM // 8):
                pltpu.make_async_copy(
                    out_hbm.at[0, i, 0],
                    transs[buf].at[pl.ds(8 * i, 8), :],
                    osems[buf],
                ).wait()

        rots = [
            jnp.bitwise_and(iota + j, LANES - 1) for j in range(LANES)
        ]

        def extract(buf):
            def kg_body(kg, carry):
                k0 = kg * LANES
                rowv = k0 + iota
                for e0 in range(0, EMBED_DIM, LANES):
                    for j0 in range(0, LANES, 8):
                        ervs = [e0 + rots[j0 + q] for q in range(8)]
                        vals = [
                            plsc.load_gather(rows[buf], [rowv, ervs[q]])
                            for q in range(8)
                        ]
                        for q in range(8):
                            plsc.store_scatter(
                                transs[buf], [ervs[q], rowv], vals[q]
                            )
                return carry

            lax.fori_loop(0, CHUNK // LANES, kg_body, 0, unroll=False)

        def write_out(buf, s, c):
            for i in range(EMBED_DIM // 8):
                pltpu.async_copy(
                    transs[buf].at[pl.ds(8 * i, 8), :],
                    out_hbm.at[s, i, j0 + c],
                    osems[buf],
                )

        pltpu.sync_copy(idxt_hbm.at[:, pl.ds(b0, b_per_w)], idxall)

        build_and_fire(0, 0, 0)

        def pair_body(t2, carry):
            for buf in range(2):
                t = 2 * t2 + buf
                s = t // n_chunks
                c = lax.rem(t, n_chunks)
                tn = t + 1
                sn = tn // n_chunks
                cn = lax.rem(tn, n_chunks)

                @pl.when(t < n_steps - 1)
                def _():
                    build_and_fire(1 - buf, sn, cn)

                drain_gather(buf)

                @pl.when(t2 >= 1)
                def _():
                    wait_out(buf)

                extract(buf)
                write_out(buf, s, c)
            return carry

        lax.fori_loop(0, n_steps // 2, pair_body, 0, unroll=False)

        wait_out(0)
        wait_out(1)

    return gather_kernel


def kernel(target_index, embedding_table):
    batch = target_index.shape[0]
    skey = jax.random.key(42)
    sampled_idx = jax.random.randint(
        skey, (batch, NUM_SAMPLED), 1, VOCAB_SIZE, dtype=jnp.int32
    )
    idxt = jnp.transpose(sampled_idx)
    tablep = jnp.pad(embedding_table, ((0, 0), (0, EMBED_DIM))).reshape(
        2 * VOCAB_SIZE, EMBED_DIM
    )
    gather_kernel = _make_gather(batch)
    out5 = gather_kernel(tablep, idxt)
    return jnp.transpose(out5, (2, 4, 0, 1, 3)).reshape(
        batch, NUM_SAMPLED, EMBED_DIM
    )


# speedup vs baseline: 3.3784x; 1.2680x over previous
import functools

import jax
import jax.numpy as jnp
from jax import lax
from jax.experimental import pallas as pl
from jax.experimental.pallas import tpu as pltpu
from jax.experimental.pallas import tpu_sc as plsc

VOCAB_SIZE = 1000000
EMBED_DIM = 64
NUM_SAMPLED = 64

CHUNK = 128
LANES = 16


@functools.lru_cache(maxsize=None)
def _make_gather(batch):
    info = plsc.get_sparse_core_info()
    nc, ns = info.num_cores, info.num_subcores
    nw = nc * ns
    b_per_w = batch // nw
    assert b_per_w % CHUNK == 0
    n_chunks = b_per_w // CHUNK
    n_steps = NUM_SAMPLED * n_chunks
    n_bj = batch // CHUNK

    mesh = plsc.VectorSubcoreMesh(core_axis_name="c", subcore_axis_name="s")

    @functools.partial(
        pl.kernel,
        mesh=mesh,
        out_type=jax.ShapeDtypeStruct(
            (NUM_SAMPLED, EMBED_DIM // 8, n_bj, 8, CHUNK), jnp.float32
        ),
        compiler_params=pltpu.CompilerParams(
            use_tc_tiling_on_sc=False, needs_layout_passes=False
        ),
        scratch_types=[
            pltpu.VMEM((NUM_SAMPLED, b_per_w), jnp.int32),
            pltpu.VMEM((CHUNK,), jnp.int32),
            pltpu.VMEM((CHUNK,), jnp.int32),
            pltpu.VMEM((CHUNK, EMBED_DIM), jnp.float32),
            pltpu.VMEM((CHUNK, EMBED_DIM), jnp.float32),
            pltpu.VMEM((EMBED_DIM, CHUNK), jnp.float32),
            pltpu.VMEM((EMBED_DIM, CHUNK), jnp.float32),
            pltpu.SemaphoreType.DMA,
            pltpu.SemaphoreType.DMA,
            pltpu.SemaphoreType.DMA,
            pltpu.SemaphoreType.DMA,
        ],
    )
    def gather_kernel(
        tablep_hbm, idxt_hbm, out_hbm,
        idxall, pr0, pr1, rows0, rows1, trans0, trans1,
        gs0, gs1, os0, os1,
    ):
        wid = lax.axis_index("s") * nc + lax.axis_index("c")
        b0 = wid * b_per_w
        j0 = wid * n_chunks
        prs = (pr0, pr1)
        rows = (rows0, rows1)
        transs = (trans0, trans1)
        gsems = (gs0, gs1)
        osems = (os0, os1)
        iota = lax.iota(jnp.int32, LANES)

        def build_and_fire(buf, s, c):
            for g in range(CHUNK // LANES):
                iv = idxall[s, pl.ds(c * CHUNK + g * LANES, LANES)]
                prs[buf][pl.ds(g * LANES, LANES)] = lax.shift_left(iv, 1)
            pltpu.async_copy(tablep_hbm.at[prs[buf]], rows[buf], gsems[buf])

        def drain_gather(buf):
            pltpu.make_async_copy(
                tablep_hbm.at[pl.ds(0, CHUNK)], rows[buf], gsems[buf]
            ).wait()

        def wait_out(buf):
            for i in range(EMBED_DIM // 8):
                pltpu.make_async_copy(
                    out_hbm.at[0, i, 0],
                    transs[buf].at[pl.ds(8 * i, 8), :],
                    osems[buf],
                ).wait()

        rots = [
            jnp.bitwise_and(iota + j, LANES - 1) for j in range(LANES)
        ]

        def extract(buf):
            def kg_body(kg, carry):
                k0 = kg * LANES
                rowv = k0 + iota
                for e0 in range(0, EMBED_DIM, LANES):
                    for j0 in range(0, LANES, 8):
                        ervs = [e0 + rots[j0 + q] for q in range(8)]
                        vals = [
                            plsc.load_gather(rows[buf], [rowv, ervs[q]])
                            for q in range(8)
                        ]
                        for q in range(8):
                            plsc.store_scatter(
                                transs[buf], [ervs[q], rowv], vals[q]
                            )
                return carry

            lax.fori_loop(0, CHUNK // LANES, kg_body, 0, unroll=False)

        def write_out(buf, s, c):
            for i in range(EMBED_DIM // 8):
                pltpu.async_copy(
                    transs[buf].at[pl.ds(8 * i, 8), :],
                    out_hbm.at[s, i, j0 + c],
                    osems[buf],
                )

        pltpu.sync_copy(idxt_hbm.at[:, pl.ds(b0, b_per_w)], idxall)

        build_and_fire(0, 0, 0)

        def pair_body(t2, carry):
            for buf in range(2):
                t = 2 * t2 + buf
                s = t // n_chunks
                c = lax.rem(t, n_chunks)
                tn = t + 1
                sn = tn // n_chunks
                cn = lax.rem(tn, n_chunks)

                @pl.when(t < n_steps - 1)
                def _():
                    build_and_fire(1 - buf, sn, cn)

                drain_gather(buf)

                @pl.when(t2 >= 1)
                def _():
                    wait_out(buf)

                extract(buf)
                write_out(buf, s, c)
            return carry

        lax.fori_loop(0, n_steps // 2, pair_body, 0, unroll=False)

        wait_out(0)
        wait_out(1)

    return gather_kernel


@functools.lru_cache(maxsize=None)
def _make_padder():
    info = plsc.get_sparse_core_info()
    nc, ns = info.num_cores, info.num_subcores
    nw = nc * ns
    n_blocks = VOCAB_SIZE // 128
    blocks_per_w = (n_blocks + nw - 1) // nw
    tail_r0 = n_blocks * 128
    tail_n = VOCAB_SIZE - tail_r0

    mesh = plsc.VectorSubcoreMesh(core_axis_name="c", subcore_axis_name="s")

    @functools.partial(
        pl.kernel,
        mesh=mesh,
        out_type=jax.ShapeDtypeStruct((VOCAB_SIZE, 2 * EMBED_DIM), jnp.float32),
        compiler_params=pltpu.CompilerParams(
            use_tc_tiling_on_sc=True, needs_layout_passes=False
        ),
        scratch_types=[
            pltpu.VMEM((EMBED_DIM, 128), jnp.float32),
            pltpu.VMEM((EMBED_DIM, 128), jnp.float32),
            pltpu.VMEM((128, 2 * EMBED_DIM), jnp.float32),
            pltpu.VMEM((128, 2 * EMBED_DIM), jnp.float32),
            pltpu.SemaphoreType.DMA,
            pltpu.SemaphoreType.DMA,
            pltpu.SemaphoreType.DMA,
            pltpu.SemaphoreType.DMA,
        ],
    )
    def pad_kernel(
        tablet_hbm, out_hbm, in0, in1, ob0, ob1, is0, is1, ws0, ws1,
    ):
        wid = lax.axis_index("s") * nc + lax.axis_index("c")
        ins = (in0, in1)
        obs = (ob0, ob1)
        isems = (is0, is1)
        wsems = (ws0, ws1)
        iota = lax.iota(jnp.int32, LANES)
        rots = [jnp.bitwise_and(iota + j, LANES - 1) for j in range(LANES)]

        def r_of(t):
            g = wid * blocks_per_w + t
            return pl.multiple_of(g * 128, 128)

        def valid(t):
            return jnp.logical_and(
                t < blocks_per_w, wid * blocks_per_w + t < n_blocks
            )

        def fire_in(buf, t):
            pltpu.async_copy(
                tablet_hbm.at[:, pl.ds(r_of(t), 128)], ins[buf], isems[buf]
            )

        def drain_in(buf):
            pltpu.make_async_copy(
                tablet_hbm.at[:, pl.ds(0, 128)], ins[buf], isems[buf]
            ).wait()

        def wait_write(buf):
            pltpu.make_async_copy(
                out_hbm.at[pl.ds(0, 128)], obs[buf], wsems[buf]
            ).wait()

        def transpose(buf):
            def kg_body(kg, carry):
                k0 = kg * LANES
                rowv = k0 + iota
                for e0 in range(0, EMBED_DIM, LANES):
                    for j0 in range(0, LANES, 8):
                        ervs = [e0 + rots[j0 + q] for q in range(8)]
                        vals = [
                            plsc.load_gather(ins[buf], [ervs[q], rowv])
                            for q in range(8)
                        ]
                        for q in range(8):
                            plsc.store_scatter(
                                obs[buf], [rowv, ervs[q]], vals[q]
                            )
                return carry

            lax.fori_loop(0, 128 // LANES, kg_body, 0, unroll=False)

        @pl.when(valid(0))
        def _():
            fire_in(0, 0)

        def pair_body(t2, carry):
            for buf in range(2):
                t = 2 * t2 + buf

                @pl.when(valid(t))
                def _():
                    @pl.when(valid(t + 1))
                    def _():
                        fire_in(1 - buf, t + 1)

                    drain_in(buf)

                    @pl.when(t2 >= 1)
                    def _():
                        wait_write(buf)

                    transpose(buf)
                    pltpu.async_copy(
                        obs[buf], out_hbm.at[pl.ds(r_of(t), 128)], wsems[buf]
                    )
            return carry

        n_pairs = (blocks_per_w + 2) // 2
        lax.fori_loop(0, n_pairs, pair_body, 0, unroll=False)

        @pl.when(valid(0))
        def _():
            wait_write(0)

        @pl.when(valid(1))
        def _():
            wait_write(1)

    return pad_kernel


def kernel(target_index, embedding_table):
    batch = target_index.shape[0]
    skey = jax.random.key(42)
    sampled_idx = jax.random.randint(
        skey, (batch, NUM_SAMPLED), 1, VOCAB_SIZE, dtype=jnp.int32
    )
    idxt = jnp.transpose(sampled_idx)
    tablet = jnp.transpose(embedding_table)
    padded = _make_padder()(tablet)
    tail_r0 = (VOCAB_SIZE // 128) * 128
    tail = jnp.pad(
        embedding_table[tail_r0:], ((0, 0), (0, EMBED_DIM))
    )
    padded = lax.dynamic_update_slice(padded, tail, (tail_r0, 0))
    tablep = padded.reshape(2 * VOCAB_SIZE, EMBED_DIM)
    gather_kernel = _make_gather(batch)
    out5 = gather_kernel(tablep, idxt)
    return jnp.transpose(out5, (2, 4, 0, 1, 3)).reshape(
        batch, NUM_SAMPLED, EMBED_DIM
    )
